# baseline (speedup 1.0000x reference)
import functools

import jax
import jax.numpy as jnp
from jax import lax
from jax.experimental import pallas as pl
from jax.experimental.pallas import tpu as pltpu
from jax.experimental.pallas import tpu_sc as plsc

B = 16384
D = 32

_info = plsc.get_sparse_core_info()
NC, NS, L = _info.num_cores, _info.num_subcores, _info.num_lanes
NW = NC * NS
BPW = B // NW
CHUNK = 128
NCHUNK = BPW // CHUNK


def _mf_body(bu_hbm, bi_hbm, p_hbm, q_hbm, out_hbm,
             idx_u, idx_i, p_v, q_v, out_v, sem):
    wid = lax.axis_index("s") * NC + lax.axis_index("c")
    base = wid * BPW
    pltpu.sync_copy(bu_hbm.at[pl.ds(base, BPW)], idx_u)
    pltpu.sync_copy(bi_hbm.at[pl.ds(base, BPW)], idx_i)

    handles = []
    for c in range(NCHUNK):
        sl = pl.ds(c * CHUNK, CHUNK)
        handles.append(pltpu.async_copy(p_hbm.at[idx_u.at[sl]], p_v.at[sl], sem))
        handles.append(pltpu.async_copy(q_hbm.at[idx_i.at[sl]], q_v.at[sl], sem))
    for h in handles:
        h.wait()

    lane = lax.iota(jnp.int32, L)
    perms = [lane ^ o for o in (8, 4, 2, 1)]

    def permute(v, perm):
        return lax.gather(
            v, perm[:, None],
            lax.GatherDimensionNumbers(
                offset_dims=(), collapsed_slice_dims=(0,), start_index_map=(0,)),
            slice_sizes=(1,),
            mode=lax.GatherScatterMode.PROMISE_IN_BOUNDS)

    def group(g, carry):
        acc = jnp.zeros((L,), jnp.float32)
        for r in range(L):
            b = g * L + r
            p0 = p_v[b, pl.ds(0, L)]
            p1 = p_v[b, pl.ds(L, L)]
            q0 = q_v[b, pl.ds(0, L)]
            q1 = q_v[b, pl.ds(L, L)]
            prod = p0 * q0 + p1 * q1
            for perm in perms:
                prod = prod + permute(prod, perm)
            acc = jnp.where(lane == r, prod, acc)
        out_v[pl.ds(g * L, L)] = acc
        return carry

    lax.fori_loop(0, BPW // L, group, 0)
    pltpu.sync_copy(out_v, out_hbm.at[pl.ds(base, BPW)])


@functools.partial(
    pl.kernel,
    mesh=plsc.VectorSubcoreMesh(core_axis_name="c", subcore_axis_name="s"),
    out_type=jax.ShapeDtypeStruct((B,), jnp.float32),
    scratch_types=[
        pltpu.VMEM((BPW,), jnp.int32),
        pltpu.VMEM((BPW,), jnp.int32),
        pltpu.VMEM((BPW, D), jnp.float32),
        pltpu.VMEM((BPW, D), jnp.float32),
        pltpu.VMEM((BPW,), jnp.float32),
        pltpu.SemaphoreType.DMA,
    ],
    compiler_params=pltpu.CompilerParams(use_tc_tiling_on_sc=False),
)
def _mf_sc(bu, bi, p, q, out, *scratch):
    _mf_body(bu, bi, p, q, out, *scratch)


def kernel(b_users, b_items, P, Q):
    out = _mf_sc(b_users.astype(jnp.int32), b_items.astype(jnp.int32), P, Q)
    return out[:, None]
